# Initial kernel scaffold; baseline (speedup 1.0000x reference)
import jax
import jax.numpy as jnp
from jax.experimental import pallas as pl


def kernel(H):
    raise NotImplementedError("write your pallas kernel here")


# speedup vs baseline: 1.7012x; 1.7012x over previous
import jax
import jax.numpy as jnp
from jax import lax
from jax.experimental import pallas as pl
from jax.experimental.pallas import tpu as pltpu

_N = 64
_K = 3072
_F = 4096


def _corr_body(x_ref, nf_ref, corr_ref):
    x = x_ref[0]
    nf_ref[0] = x
    mean = jnp.mean(x, axis=-1, keepdims=True)
    xc = x - mean
    var = jnp.sum(xc * xc, axis=-1, keepdims=True) / (_F - 1)
    std = jnp.sqrt(var) + 1e-8
    xn = xc / std
    corr = lax.dot_general(xn, xn, (((1,), (1,)), ((), ())),
                           preferred_element_type=jnp.float32)
    corr_ref[0] = corr / _F


def _select_body(corr_ref, adj_ref):
    c = corr_ref[...]
    b = lax.bitcast_convert_type(c, jnp.int32)
    skey = b ^ ((b >> 31) & jnp.int32(0x7FFFFFFF))

    def _count(mask):
        m = mask.astype(jnp.int32)
        return jnp.sum(jnp.sum(m, axis=-1, keepdims=True), axis=-2, keepdims=True)

    neg = skey < 0
    cnt_neg = _count(neg)
    sign_neg = cnt_neg >= _K
    kp = jnp.where(sign_neg, _K, _K - cnt_neg)
    cand = neg == sign_neg
    low31 = skey & jnp.int32(0x7FFFFFFF)

    prefix = jnp.zeros(cnt_neg.shape, jnp.int32)
    for bit in range(30, -1, -1):
        trial = prefix | jnp.int32(1 << bit)
        cnt = _count(cand & (low31 < trial))
        prefix = jnp.where(cnt >= kp, prefix, trial)

    skey_thr = jnp.where(sign_neg, prefix | jnp.int32(-2**31), prefix)
    adj = (skey > skey_thr).astype(jnp.float32)
    row = lax.broadcasted_iota(jnp.int32, c.shape, 1)
    col = lax.broadcasted_iota(jnp.int32, c.shape, 2)
    adj_ref[...] = jnp.where(row == col, 0.0, adj)


def kernel(H):
    B, S, Hd = H.shape
    X = H.reshape(B, _N, _F)
    nf, corr = pl.pallas_call(
        _corr_body,
        grid=(B,),
        in_specs=[pl.BlockSpec((1, _N, _F), lambda i: (i, 0, 0))],
        out_specs=[
            pl.BlockSpec((1, _N, _F), lambda i: (i, 0, 0)),
            pl.BlockSpec((1, _N, _N), lambda i: (i, 0, 0)),
        ],
        out_shape=[
            jax.ShapeDtypeStruct((B, _N, _F), jnp.float32),
            jax.ShapeDtypeStruct((B, _N, _N), jnp.float32),
        ],
    )(X)
    adj = pl.pallas_call(
        _select_body,
        out_shape=jax.ShapeDtypeStruct((B, _N, _N), jnp.float32),
    )(corr)
    return (nf, adj)
